# Initial kernel scaffold; baseline (speedup 1.0000x reference)
import jax
import jax.numpy as jnp
from jax.experimental import pallas as pl


def kernel(x, edge_index, angles, transporters, K_self, K_neigh_w, ln_gamma, ln_beta):
    raise NotImplementedError("write your pallas kernel here")


# speedup vs baseline: 2.5674x; 2.5674x over previous
import functools

import numpy as _np

import jax
import jax.numpy as jnp
from jax import lax
from jax.experimental import pallas as pl
from jax.experimental.pallas import tpu as pltpu
from jax.experimental.pallas import tpu_sc as plsc

V = 10000
E = 160000
DIM = 24
M0 = 8
NB = 5
NC = 2
NS = 16
NW = NC * NS
EPW = E // NW
CH = 125
NCH = EPW // CH
VPW = V // NS
SCCH = 1250

@functools.cache
def _sc_mesh():
    return plsc.VectorSubcoreMesh(core_axis_name="c", subcore_axis_name="s")


def _gather_body(x_hbm, src_hbm, out_hbm, idx_v, rows_v, sem):
    c = lax.axis_index("c")
    s = lax.axis_index("s")
    wid = c * NS + s
    pltpu.sync_copy(src_hbm.at[pl.ds(wid * NCH, NCH)], idx_v)

    def chunk_group(g, _):
        descs = []
        for j in range(8):
            k = g * 8 + j
            descs.append(
                pltpu.async_copy(
                    x_hbm.at[idx_v.at[k]],
                    rows_v.at[pl.ds(k * CH, CH)],
                    sem,
                )
            )
        for d in descs:
            d.wait()
        return _

    lax.fori_loop(0, NCH // 8, chunk_group, 0)
    pltpu.sync_copy(rows_v, out_hbm.at[pl.ds(wid * EPW, EPW)])


@jax.jit
def _gather(x, src2d):
    return pl.kernel(
        _gather_body,
        out_type=jax.ShapeDtypeStruct((E, DIM), jnp.float32),
        mesh=_sc_mesh(),
        scratch_types=[
            pltpu.VMEM((NCH, CH), jnp.int32),
            pltpu.VMEM((EPW, DIM), jnp.float32),
            pltpu.SemaphoreType.DMA,
        ],
        compiler_params=pltpu.CompilerParams(use_tc_tiling_on_sc=False),
    )(x, src2d)


TB = 1280


def _trig_body(a_ref, t_ref, ct_ref, st_ref, ca_ref, sa_ref, c2_ref, s2_ref):
    a = a_ref[...]
    t = t_ref[...]
    ct_ref[...] = jnp.cos(t)
    st_ref[...] = jnp.sin(t)
    ca = jnp.cos(a)
    sa = jnp.sin(a)
    ca_ref[...] = ca
    sa_ref[...] = sa
    c2_ref[...] = 2.0 * ca * ca - 1.0
    s2_ref[...] = 2.0 * sa * ca


@jax.jit
def _trig(ang, trn):
    a2 = ang.reshape(E // TB, TB)
    t2 = trn.reshape(E // TB, TB)
    outs = pl.pallas_call(
        _trig_body,
        out_shape=[jax.ShapeDtypeStruct((E // TB, TB), jnp.float32)] * 6,
    )(a2, t2)
    flat = [o.reshape(E) for o in outs]
    one = jnp.ones((E,), jnp.float32)
    return jnp.stack([one, *flat, one], axis=1)


BE = 2000


def _edge_body(xs_ref, c8_ref, a12_ref, a3_ref, w2_ref, red_ref, msg_ref):
    xs = xs_ref[...]
    c8 = c8_ref[...]
    mm = jnp.dot(c8, a12_ref[...], preferred_element_type=jnp.float32)
    cexp = jnp.dot(c8, a3_ref[...], preferred_element_type=jnp.float32)
    z = jnp.concatenate([xs, xs], axis=1) * mm
    g = jnp.dot(z, w2_ref[...], preferred_element_type=jnp.float32)
    msg_ref[...] = jnp.dot(g * cexp, red_ref[...], preferred_element_type=jnp.float32)


@jax.jit
def _edge_compute(xs, c8, a12, a3, w2, red):
    grid = E // BE
    return pl.pallas_call(
        _edge_body,
        grid=(grid,),
        in_specs=[
            pl.BlockSpec((BE, DIM), lambda i: (i, 0)),
            pl.BlockSpec((BE, 8), lambda i: (i, 0)),
            pl.BlockSpec((8, 2 * DIM), lambda i: (0, 0)),
            pl.BlockSpec((8, NB * DIM), lambda i: (0, 0)),
            pl.BlockSpec((2 * DIM, NB * DIM), lambda i: (0, 0)),
            pl.BlockSpec((NB * DIM, DIM), lambda i: (0, 0)),
        ],
        out_specs=pl.BlockSpec((BE, DIM), lambda i: (i, 0)),
        out_shape=jax.ShapeDtypeStruct((E, DIM), jnp.float32),
    )(xs, c8, a12, a3, w2, red)


def _scatter_body(msg_hbm, tgt_hbm, zeros_hbm, out_hbm, idx_v, rows_v, sem, acc_sh):
    c = lax.axis_index("c")
    s = lax.axis_index("s")
    wid = c * NS + s
    pltpu.sync_copy(zeros_hbm, acc_sh.at[pl.ds(s * VPW, VPW)])
    pltpu.sync_copy(tgt_hbm.at[pl.ds(wid * NCH, NCH)], idx_v)
    plsc.subcore_barrier()

    def super_chunk(g, _):
        pltpu.sync_copy(msg_hbm.at[pl.ds(wid * EPW + g * SCCH, SCCH)], rows_v)
        descs = []
        for j in range(SCCH // CH):
            descs.append(
                pltpu.async_copy(
                    rows_v.at[pl.ds(j * CH, CH)],
                    acc_sh.at[idx_v.at[g * (SCCH // CH) + j]],
                    sem,
                    add=True,
                )
            )
        for d in descs:
            d.wait()
        return _

    lax.fori_loop(0, EPW // SCCH, super_chunk, 0)
    plsc.subcore_barrier()
    pltpu.sync_copy(
        acc_sh.at[pl.ds(s * VPW, VPW)],
        out_hbm.at[pl.ds(c * V + s * VPW, VPW)],
    )


@jax.jit
def _scatter(msg, tgt2d, zeros):
    return pl.kernel(
        _scatter_body,
        out_type=jax.ShapeDtypeStruct((NC * V, DIM), jnp.float32),
        mesh=_sc_mesh(),
        scratch_types=[
            pltpu.VMEM((NCH, CH), jnp.int32),
            pltpu.VMEM((SCCH, DIM), jnp.float32),
            pltpu.SemaphoreType.DMA,
            pltpu.VMEM_SHARED((V, DIM), jnp.float32),
        ],
        compiler_params=pltpu.CompilerParams(use_tc_tiling_on_sc=False),
    )(msg, tgt2d, zeros)


BV = 2000


def _final_body(x_ref, p0_ref, p1_ref, ks_ref, g_ref, b_ref, out_ref):
    x = x_ref[...]
    h = (
        jnp.dot(x, ks_ref[...], preferred_element_type=jnp.float32)
        + p0_ref[...]
        + p1_ref[...]
    )
    mu = jnp.mean(h, axis=1, keepdims=True)
    d = h - mu
    var = jnp.mean(d * d, axis=1, keepdims=True)
    hn = d * lax.rsqrt(var + 1e-5) * g_ref[...] + b_ref[...]
    lane = lax.broadcasted_iota(jnp.int32, (1, DIM), 1)
    is_scal = lane < M0
    is_x = jnp.logical_and(lane >= M0, lane % 2 == 0)
    n2 = hn * hn
    psum = n2 + jnp.where(is_x, jnp.roll(n2, -1, axis=1), jnp.roll(n2, 1, axis=1))
    norm = jnp.maximum(jnp.sqrt(psum), 1e-8)
    scale = jax.nn.softplus(norm) / norm
    out_ref[...] = jnp.where(is_scal, jnp.maximum(hn, 0.0), hn * scale) + x


@jax.jit
def _final(x, p0, p1, ks_t, gamma, beta):
    grid = V // BV
    return pl.pallas_call(
        _final_body,
        grid=(grid,),
        in_specs=[
            pl.BlockSpec((BV, DIM), lambda i: (i, 0)),
            pl.BlockSpec((BV, DIM), lambda i: (i, 0)),
            pl.BlockSpec((BV, DIM), lambda i: (i, 0)),
            pl.BlockSpec((DIM, DIM), lambda i: (0, 0)),
            pl.BlockSpec((1, DIM), lambda i: (0, 0)),
            pl.BlockSpec((1, DIM), lambda i: (0, 0)),
        ],
        out_specs=pl.BlockSpec((BV, DIM), lambda i: (i, 0)),
        out_shape=jax.ShapeDtypeStruct((V, DIM), jnp.float32),
    )(x, p0, p1, ks_t, gamma, beta)


def kernel(x, edge_index, angles, transporters, K_self, K_neigh_w, ln_gamma, ln_beta):
    src2d = edge_index[0].astype(jnp.int32).reshape(NW * NCH, CH)
    tgt2d = edge_index[1].astype(jnp.int32).reshape(NW * NCH, CH)
    xs = _gather(x, src2d)
    wt = jnp.transpose(K_neigh_w, (2, 0, 1)).reshape(DIM, NB * DIM)
    perm = _np.arange(DIM)
    perm[M0:] = perm[M0:].reshape(-1, 2)[:, ::-1].ravel()
    w2 = jnp.concatenate([wt, wt[perm]], axis=0)
    red = jnp.tile(jnp.eye(DIM, dtype=jnp.float32), (NB, 1))
    a12 = _np.zeros((8, 2 * DIM), dtype=_np.float32)
    a12[0, :M0] = 1.0
    a12[1, M0:DIM] = 1.0
    for m in range(M0, DIM, 2):
        a12[2, DIM + m] = 1.0
        a12[2, DIM + m + 1] = -1.0
    a3 = _np.zeros((8, NB * DIM), dtype=_np.float32)
    for b, row in enumerate((0, 3, 4, 5, 6)):
        a3[row, b * DIM:(b + 1) * DIM] = 1.0
    c8 = _trig(angles, transporters)
    msg = _edge_compute(xs, c8, jnp.asarray(a12), jnp.asarray(a3), w2, red)
    zeros = jnp.zeros((VPW, DIM), jnp.float32)
    partial = _scatter(msg, tgt2d, zeros)
    return _final(
        x,
        partial[:V],
        partial[V:],
        K_self.T,
        ln_gamma.reshape(1, DIM),
        ln_beta.reshape(1, DIM),
    )


# speedup vs baseline: 2.8672x; 1.1168x over previous
import functools

import numpy as _np

import jax
import jax.numpy as jnp
from jax import lax
from jax.experimental import pallas as pl
from jax.experimental.pallas import tpu as pltpu
from jax.experimental.pallas import tpu_sc as plsc

V = 10000
E = 160000
DIM = 24
M0 = 8
NB = 5
NC = 2
NS = 16
NW = NC * NS
EPW = E // NW
CH = 125
NCH = EPW // CH
VPW = V // NS
SCCH = 1250

@functools.cache
def _sc_mesh():
    return plsc.VectorSubcoreMesh(core_axis_name="c", subcore_axis_name="s")


GCH = 128
NGCH = EPW // GCH
GREM = EPW - NGCH * GCH


def _gather_body(x_hbm, ei_hbm, out_hbm, idx_v, rows_v, sem):
    c = lax.axis_index("c")
    s = lax.axis_index("s")
    wid = c * NS + s
    pltpu.sync_copy(ei_hbm.at[0, pl.ds(wid * EPW, EPW)], idx_v)
    descs = []
    for k in range(NGCH):
        descs.append(
            pltpu.async_copy(
                x_hbm.at[idx_v.at[pl.ds(k * GCH, GCH)]],
                rows_v.at[pl.ds(k * GCH, GCH)],
                sem,
            )
        )
    descs.append(
        pltpu.async_copy(
            x_hbm.at[idx_v.at[pl.ds(NGCH * GCH, GREM)]],
            rows_v.at[pl.ds(NGCH * GCH, GREM)],
            sem,
        )
    )
    for d in descs:
        d.wait()
    pltpu.sync_copy(rows_v, out_hbm.at[pl.ds(wid * EPW, EPW)])


@jax.jit
def _gather(x, ei32):
    return pl.kernel(
        _gather_body,
        out_type=jax.ShapeDtypeStruct((E, DIM), jnp.float32),
        mesh=_sc_mesh(),
        scratch_types=[
            pltpu.VMEM((EPW,), jnp.int32),
            pltpu.VMEM((EPW, DIM), jnp.float32),
            pltpu.SemaphoreType.DMA,
        ],
        compiler_params=pltpu.CompilerParams(use_tc_tiling_on_sc=False),
    )(x, ei32)


TB = 1280


def _trig_body(a_ref, t_ref, ct_ref, st_ref, ca_ref, sa_ref, c2_ref, s2_ref):
    a = a_ref[...]
    t = t_ref[...]
    ct_ref[...] = jnp.cos(t)
    st_ref[...] = jnp.sin(t)
    ca = jnp.cos(a)
    sa = jnp.sin(a)
    ca_ref[...] = ca
    sa_ref[...] = sa
    c2_ref[...] = 2.0 * ca * ca - 1.0
    s2_ref[...] = 2.0 * sa * ca


@jax.jit
def _trig(ang, trn):
    a2 = ang.reshape(E // TB, TB)
    t2 = trn.reshape(E // TB, TB)
    outs = pl.pallas_call(
        _trig_body,
        out_shape=[jax.ShapeDtypeStruct((E // TB, TB), jnp.float32)] * 6,
    )(a2, t2)
    flat = [o.reshape(E) for o in outs]
    one = jnp.ones((E,), jnp.float32)
    return jnp.stack([one, *flat, one], axis=1)


BE = 8000


def _edge_body(xs_ref, c8_ref, a12_ref, a3_ref, w2_ref, red_ref, msg_ref):
    xs = xs_ref[...]
    c8 = c8_ref[...]
    mm = jnp.dot(c8, a12_ref[...], preferred_element_type=jnp.float32)
    cexp = jnp.dot(c8, a3_ref[...], preferred_element_type=jnp.float32)
    z = jnp.concatenate([xs, xs], axis=1) * mm
    g = jnp.dot(z, w2_ref[...], preferred_element_type=jnp.float32)
    msg_ref[...] = jnp.dot(g * cexp, red_ref[...], preferred_element_type=jnp.float32)


@jax.jit
def _edge_compute(xs, c8, a12, a3, w2, red):
    grid = E // BE
    return pl.pallas_call(
        _edge_body,
        grid=(grid,),
        in_specs=[
            pl.BlockSpec((BE, DIM), lambda i: (i, 0)),
            pl.BlockSpec((BE, 8), lambda i: (i, 0)),
            pl.BlockSpec((8, 2 * DIM), lambda i: (0, 0)),
            pl.BlockSpec((8, NB * DIM), lambda i: (0, 0)),
            pl.BlockSpec((2 * DIM, NB * DIM), lambda i: (0, 0)),
            pl.BlockSpec((NB * DIM, DIM), lambda i: (0, 0)),
        ],
        out_specs=pl.BlockSpec((BE, DIM), lambda i: (i, 0)),
        out_shape=jax.ShapeDtypeStruct((E, DIM), jnp.float32),
    )(xs, c8, a12, a3, w2, red)


SCH = 120
NSCH = EPW // SCH
SREM = EPW - NSCH * SCH
SSC = 1200
NSSC = 4


def _scatter_body(msg_hbm, ei_hbm, zeros_hbm, out_hbm, idx_v, rows_v, sem, acc_sh):
    c = lax.axis_index("c")
    s = lax.axis_index("s")
    wid = c * NS + s
    pltpu.sync_copy(zeros_hbm, acc_sh.at[pl.ds(s * VPW, VPW)])
    idescs = []
    for k in range(NSCH):
        idescs.append(
            pltpu.async_copy(
                ei_hbm.at[1, pl.ds(wid * EPW + k * SCH, SCH)],
                idx_v.at[k],
                sem,
            )
        )
    idescs.append(
        pltpu.async_copy(
            ei_hbm.at[1, pl.ds(wid * EPW + NSCH * SCH, SREM)],
            idx_v.at[NSCH, pl.ds(0, SREM)],
            sem,
        )
    )
    for d in idescs:
        d.wait()
    plsc.subcore_barrier()

    for g in range(NSSC):
        pltpu.sync_copy(msg_hbm.at[pl.ds(wid * EPW + g * SSC, SSC)], rows_v)
        descs = []
        for j in range(SSC // SCH):
            descs.append(
                pltpu.async_copy(
                    rows_v.at[pl.ds(j * SCH, SCH)],
                    acc_sh.at[idx_v.at[g * (SSC // SCH) + j]],
                    sem,
                    add=True,
                )
            )
        for d in descs:
            d.wait()
    tail = EPW - NSSC * SSC
    pltpu.sync_copy(msg_hbm.at[pl.ds(wid * EPW + NSSC * SSC, tail)],
                    rows_v.at[pl.ds(0, tail)])
    descs = [
        pltpu.async_copy(
            rows_v.at[pl.ds(0, SCH)],
            acc_sh.at[idx_v.at[NSSC * (SSC // SCH)]],
            sem,
            add=True,
        ),
        pltpu.async_copy(
            rows_v.at[pl.ds(SCH, SREM)],
            acc_sh.at[idx_v.at[NSCH, pl.ds(0, SREM)]],
            sem,
            add=True,
        ),
    ]
    for d in descs:
        d.wait()
    plsc.subcore_barrier()
    pltpu.sync_copy(
        acc_sh.at[pl.ds(s * VPW, VPW)],
        out_hbm.at[pl.ds(c * V + s * VPW, VPW)],
    )


@jax.jit
def _scatter(msg, ei32, zeros):
    return pl.kernel(
        _scatter_body,
        out_type=jax.ShapeDtypeStruct((NC * V, DIM), jnp.float32),
        mesh=_sc_mesh(),
        scratch_types=[
            pltpu.VMEM((NSCH + 1, SCH), jnp.int32),
            pltpu.VMEM((SSC, DIM), jnp.float32),
            pltpu.SemaphoreType.DMA,
            pltpu.VMEM_SHARED((V, DIM), jnp.float32),
        ],
        compiler_params=pltpu.CompilerParams(use_tc_tiling_on_sc=False),
    )(msg, ei32, zeros)


BV = 2000


def _final_body(x_ref, p0_ref, p1_ref, ks_ref, g_ref, b_ref, out_ref):
    x = x_ref[...]
    h = (
        jnp.dot(x, ks_ref[...], preferred_element_type=jnp.float32)
        + p0_ref[...]
        + p1_ref[...]
    )
    mu = jnp.mean(h, axis=1, keepdims=True)
    d = h - mu
    var = jnp.mean(d * d, axis=1, keepdims=True)
    hn = d * lax.rsqrt(var + 1e-5) * g_ref[...] + b_ref[...]
    lane = lax.broadcasted_iota(jnp.int32, (1, DIM), 1)
    is_scal = lane < M0
    is_x = jnp.logical_and(lane >= M0, lane % 2 == 0)
    n2 = hn * hn
    psum = n2 + jnp.where(is_x, jnp.roll(n2, -1, axis=1), jnp.roll(n2, 1, axis=1))
    norm = jnp.maximum(jnp.sqrt(psum), 1e-8)
    scale = jax.nn.softplus(norm) / norm
    out_ref[...] = jnp.where(is_scal, jnp.maximum(hn, 0.0), hn * scale) + x


@jax.jit
def _final(x, p0, p1, ks_t, gamma, beta):
    grid = V // BV
    return pl.pallas_call(
        _final_body,
        grid=(grid,),
        in_specs=[
            pl.BlockSpec((BV, DIM), lambda i: (i, 0)),
            pl.BlockSpec((BV, DIM), lambda i: (i, 0)),
            pl.BlockSpec((BV, DIM), lambda i: (i, 0)),
            pl.BlockSpec((DIM, DIM), lambda i: (0, 0)),
            pl.BlockSpec((1, DIM), lambda i: (0, 0)),
            pl.BlockSpec((1, DIM), lambda i: (0, 0)),
        ],
        out_specs=pl.BlockSpec((BV, DIM), lambda i: (i, 0)),
        out_shape=jax.ShapeDtypeStruct((V, DIM), jnp.float32),
    )(x, p0, p1, ks_t, gamma, beta)


def kernel(x, edge_index, angles, transporters, K_self, K_neigh_w, ln_gamma, ln_beta):
    ei32 = edge_index.astype(jnp.int32)
    xs = _gather(x, ei32)
    wt = jnp.transpose(K_neigh_w, (2, 0, 1)).reshape(DIM, NB * DIM)
    perm = _np.arange(DIM)
    perm[M0:] = perm[M0:].reshape(-1, 2)[:, ::-1].ravel()
    w2 = jnp.concatenate([wt, wt[perm]], axis=0)
    red = jnp.tile(jnp.eye(DIM, dtype=jnp.float32), (NB, 1))
    a12 = _np.zeros((8, 2 * DIM), dtype=_np.float32)
    a12[0, :M0] = 1.0
    a12[1, M0:DIM] = 1.0
    for m in range(M0, DIM, 2):
        a12[2, DIM + m] = 1.0
        a12[2, DIM + m + 1] = -1.0
    a3 = _np.zeros((8, NB * DIM), dtype=_np.float32)
    for b, row in enumerate((0, 3, 4, 5, 6)):
        a3[row, b * DIM:(b + 1) * DIM] = 1.0
    c8 = _trig(angles, transporters)
    msg = _edge_compute(xs, c8, jnp.asarray(a12), jnp.asarray(a3), w2, red)
    zeros = jnp.zeros((VPW, DIM), jnp.float32)
    partial = _scatter(msg, ei32, zeros)
    return _final(
        x,
        partial[:V],
        partial[V:],
        K_self.T,
        ln_gamma.reshape(1, DIM),
        ln_beta.reshape(1, DIM),
    )
